# baseline (speedup 1.0000x reference)
import functools

import jax
import jax.numpy as jnp
from jax import lax
from jax.experimental import pallas as pl
from jax.experimental.pallas import tpu as pltpu
from jax.experimental.pallas import tpu_sc as plsc

_NC = 2
_NS = 16
_LN = 16
_CH = 32
_NEG = -1.0e9


def _silu(v):
    return v * (1.0 / (1.0 + jnp.exp(-v)))


def _h0_tables_body(x_ref, wi_ref, bi_ref, wd_ref, ws_ref, bd_ref,
                    h_ref, td_ref, ts_ref):
    a = jnp.dot(x_ref[...], wi_ref[...], preferred_element_type=jnp.float32)
    h = _silu(a + bi_ref[...])
    h_ref[...] = h
    d = jnp.dot(h, wd_ref[...], preferred_element_type=jnp.float32) + bd_ref[...]
    s = jnp.dot(h, ws_ref[...], preferred_element_type=jnp.float32)
    td_ref[0] = d[:, :256]
    td_ref[1] = d[:, 256:]
    ts_ref[0] = s[:, :256]
    ts_ref[1] = s[:, 256:]


def _update_tables_body(agg_ref, h_ref, g_ref, b_ref, wd_ref, ws_ref, bd_ref,
                        h_ref_out, td_ref, ts_ref):
    h = h_ref[...]
    agg = jnp.concatenate([agg_ref[0], agg_ref[1]], axis=-1)
    hn = _silu(agg + h) + h
    mu = jnp.mean(hn, axis=-1, keepdims=True)
    dvar = hn - mu
    var = jnp.mean(dvar * dvar, axis=-1, keepdims=True)
    hh = dvar * jax.lax.rsqrt(var + 1e-5) * g_ref[...] + b_ref[...]
    h_ref_out[...] = hh
    d = jnp.dot(hh, wd_ref[...], preferred_element_type=jnp.float32) + bd_ref[...]
    s = jnp.dot(hh, ws_ref[...], preferred_element_type=jnp.float32)
    td_ref[0] = d[:, :256]
    td_ref[1] = d[:, 256:]
    ts_ref[0] = s[:, :256]
    ts_ref[1] = s[:, 256:]


def _update_only_body(agg_ref, h_ref, g_ref, b_ref, h_ref_out):
    h = h_ref[...]
    agg = jnp.concatenate([agg_ref[0], agg_ref[1]], axis=-1)
    hn = _silu(agg + h) + h
    mu = jnp.mean(hn, axis=-1, keepdims=True)
    dvar = hn - mu
    var = jnp.mean(dvar * dvar, axis=-1, keepdims=True)
    h_ref_out[...] = dvar * jax.lax.rsqrt(var + 1e-5) * g_ref[...] + b_ref[...]


def _edgeproj_body(ea_ref, wa_ref, cc_ref, *, blk, e_real):
    i = pl.program_id(0)
    c = jnp.dot(ea_ref[...], wa_ref[...], preferred_element_type=jnp.float32)
    row = i * blk + lax.broadcasted_iota(jnp.int32, (blk, 1), 0)
    c = jnp.where(row < e_real, c, _NEG)
    cc_ref[0] = c[:, :256]
    cc_ref[1] = c[:, 256:]


def _pool_body(batch_ref, h_ref, wo_ref, bo_ref, out_ref, acc, cnt, *, nblk, blk, g):
    i = pl.program_id(0)

    @pl.when(i == 0)
    def _():
        acc[...] = jnp.zeros_like(acc)
        cnt[...] = jnp.zeros_like(cnt)

    ids = batch_ref[...]
    gids = lax.broadcasted_iota(jnp.int32, (blk, g), 1)
    oh = (ids == gids).astype(jnp.float32)
    h = h_ref[...]
    acc[...] += lax.dot_general(oh, h, (((0,), (0,)), ((), ())),
                                preferred_element_type=jnp.float32)
    cnt[...] += lax.dot_general(oh, jnp.ones_like(h), (((0,), (0,)), ((), ())),
                                preferred_element_type=jnp.float32)

    @pl.when(i == nblk - 1)
    def _():
        pooled = acc[...] / jnp.maximum(cnt[...], 1.0)
        out_ref[...] = (jnp.dot(pooled, wo_ref[...],
                                preferred_element_type=jnp.float32) + bo_ref[...])


def _edge_sc_body(td_hbm, ts_hbm, cc_hbm, dst_hbm, src_hbm, out_hbm,
                  dvec, svec, td_buf, ts_buf, cc_buf, m_buf, agg_sh,
                  sem0, sem1, sem2, *, n_pad, ep_tile, nch):
    c = lax.axis_index("c")
    s = lax.axis_index("s")

    zero = jnp.zeros((_LN,), jnp.float32)

    def zrow(r, _):
        for j in range(128 // _LN):
            m_buf[r, pl.ds(j * _LN, _LN)] = zero
        return 0

    lax.fori_loop(0, _CH, zrow, 0)

    rpt = n_pad // _NS
    base = s * rpt
    off = 0
    while off < rpt:
        sz = min(_CH, rpt - off)
        pltpu.sync_copy(m_buf.at[pl.ds(0, sz)], agg_sh.at[pl.ds(base + off, sz)])
        off += sz
    plsc.subcore_barrier()

    def chunk(t, _):
        e0 = s * ep_tile + t * _CH
        pltpu.sync_copy(dst_hbm.at[pl.ds(e0, _CH)], dvec)
        pltpu.sync_copy(src_hbm.at[pl.ds(e0, _CH)], svec)
        cpa = pltpu.async_copy(td_hbm.at[c].at[dvec], td_buf, sem0)
        cpb = pltpu.async_copy(ts_hbm.at[c].at[svec], ts_buf, sem1)
        cpc = pltpu.async_copy(cc_hbm.at[c].at[pl.ds(e0, _CH)], cc_buf, sem2)
        cpa.wait()
        cpb.wait()
        cpc.wait()

        def edge(e, _):
            for j in range(8):
                jc = j * _LN
                af = td_buf[e, pl.ds(jc, _LN)] + ts_buf[e, pl.ds(jc, _LN)] \
                    + cc_buf[e, pl.ds(jc, _LN)]
                as_ = td_buf[e, pl.ds(128 + jc, _LN)] \
                    + ts_buf[e, pl.ds(128 + jc, _LN)] \
                    + cc_buf[e, pl.ds(128 + jc, _LN)]
                gate = 1.0 / (1.0 + jnp.exp(-af))
                u = jnp.exp(-jnp.abs(as_))
                tt = u / (2.0 + u)
                t2 = tt * tt
                p = tt * (1.0 + t2 * (0.33333333 + t2 * (0.2 + t2 *
                          (0.14285714 + t2 * 0.11111111))))
                sp = jnp.maximum(as_, 0.0) + 2.0 * p
                m_buf[e, pl.ds(jc, _LN)] = gate * sp
            return 0

        lax.fori_loop(0, _CH, edge, 0)
        pltpu.sync_copy(m_buf, agg_sh.at[dvec], add=True)
        return 0

    lax.fori_loop(0, nch, chunk, 0)
    plsc.subcore_barrier()

    off = 0
    while off < rpt:
        sz = min(_CH, rpt - off)
        pltpu.sync_copy(agg_sh.at[pl.ds(base + off, sz)],
                        out_hbm.at[c].at[pl.ds(base + off, sz)])
        off += sz


def _edge_sc_call(td, ts, cc, dst_p, src_p, n_nodes, ep):
    ep_tile = ep // _NS
    nch = ep_tile // _CH
    n_pad = -(-n_nodes // (8 * _NS)) * (8 * _NS)
    mesh = plsc.VectorSubcoreMesh(core_axis_name="c", subcore_axis_name="s")
    body = functools.partial(_edge_sc_body, n_pad=n_pad,
                             ep_tile=ep_tile, nch=nch)
    agg = pl.kernel(
        body,
        out_type=jax.ShapeDtypeStruct((_NC, n_pad, 128), jnp.float32),
        mesh=mesh,
        scratch_types=[
            pltpu.VMEM((_CH,), jnp.int32),
            pltpu.VMEM((_CH,), jnp.int32),
            pltpu.VMEM((_CH, 256), jnp.float32),
            pltpu.VMEM((_CH, 256), jnp.float32),
            pltpu.VMEM((_CH, 256), jnp.float32),
            pltpu.VMEM((_CH, 128), jnp.float32),
            pltpu.VMEM_SHARED((n_pad, 128), jnp.float32),
            pltpu.SemaphoreType.DMA,
            pltpu.SemaphoreType.DMA,
            pltpu.SemaphoreType.DMA,
        ],
        name="edge_sc",
    )(td, ts, cc, dst_p, src_p)
    return agg[:, :n_nodes]


def _perm_w(wf, ws):
    return jnp.concatenate(
        [wf[:, :128], ws[:, :128], wf[:, 128:], ws[:, 128:]], axis=1)


def _perm_b(bf, bs):
    return jnp.concatenate([bf[:128], bs[:128], bf[128:], bs[128:]])[None, :]


def kernel(x, edge_attr, params, edge_index, batch):
    n, dd = x.shape
    e = edge_index.shape[1]
    h_dim = params['Wi'].shape[1]
    out_dim = params['Wo'].shape[1]
    n_layers = len(params['Wf'])
    g = 64

    ep = ((e + _NS * _CH - 1) // (_NS * _CH)) * (_NS * _CH)
    dst_p = jnp.concatenate(
        [edge_index[1], jnp.zeros((ep - e,), jnp.int32)]).astype(jnp.int32)
    src_p = jnp.concatenate(
        [edge_index[0], jnp.zeros((ep - e,), jnp.int32)]).astype(jnp.int32)
    ea_p = jnp.concatenate(
        [edge_attr, jnp.zeros((ep - e, edge_attr.shape[1]), jnp.float32)])

    wd = [_perm_w(params['Wf'][l][:h_dim], params['Ws'][l][:h_dim])
          for l in range(n_layers)]
    wsr = [_perm_w(params['Wf'][l][h_dim:2 * h_dim], params['Ws'][l][h_dim:2 * h_dim])
           for l in range(n_layers)]
    bd = [_perm_b(params['bf'][l], params['bs'][l]) for l in range(n_layers)]
    wa = [_perm_w(params['Wf'][l][2 * h_dim:], params['Ws'][l][2 * h_dim:])
          for l in range(n_layers)]

    nb = 1000
    n_grid = n // nb
    eb = next(b for b in (2048, 2000, 1600, 1024, 512, 256, 128)
              if ep % b == 0)
    e_grid = ep // eb

    f32 = jnp.float32

    h, td, ts = pl.pallas_call(
        _h0_tables_body,
        grid=(n_grid,),
        in_specs=[
            pl.BlockSpec((nb, dd), lambda i: (i, 0)),
            pl.BlockSpec((dd, h_dim), lambda i: (0, 0)),
            pl.BlockSpec((1, h_dim), lambda i: (0, 0)),
            pl.BlockSpec((h_dim, 2 * h_dim), lambda i: (0, 0)),
            pl.BlockSpec((h_dim, 2 * h_dim), lambda i: (0, 0)),
            pl.BlockSpec((1, 2 * h_dim), lambda i: (0, 0)),
        ],
        out_specs=[
            pl.BlockSpec((nb, h_dim), lambda i: (i, 0)),
            pl.BlockSpec((_NC, nb, h_dim), lambda i: (0, i, 0)),
            pl.BlockSpec((_NC, nb, h_dim), lambda i: (0, i, 0)),
        ],
        out_shape=[
            jax.ShapeDtypeStruct((n, h_dim), f32),
            jax.ShapeDtypeStruct((_NC, n, h_dim), f32),
            jax.ShapeDtypeStruct((_NC, n, h_dim), f32),
        ],
    )(x, params['Wi'], params['bi'][None, :], wd[0], wsr[0], bd[0])

    for l in range(n_layers):
        cc = pl.pallas_call(
            functools.partial(_edgeproj_body, blk=eb, e_real=e),
            grid=(e_grid,),
            in_specs=[
                pl.BlockSpec((eb, ea_p.shape[1]), lambda i: (i, 0)),
                pl.BlockSpec((ea_p.shape[1], 2 * h_dim), lambda i: (0, 0)),
            ],
            out_specs=pl.BlockSpec((_NC, eb, h_dim), lambda i: (0, i, 0)),
            out_shape=jax.ShapeDtypeStruct((_NC, ep, h_dim), f32),
        )(ea_p, wa[l])

        agg = _edge_sc_call(td, ts, cc, dst_p, src_p, n, ep)

        if l + 1 < n_layers:
            h, td, ts = pl.pallas_call(
                _update_tables_body,
                grid=(n_grid,),
                in_specs=[
                    pl.BlockSpec((_NC, nb, 128), lambda i: (0, i, 0)),
                    pl.BlockSpec((nb, h_dim), lambda i: (i, 0)),
                    pl.BlockSpec((1, h_dim), lambda i: (0, 0)),
                    pl.BlockSpec((1, h_dim), lambda i: (0, 0)),
                    pl.BlockSpec((h_dim, 2 * h_dim), lambda i: (0, 0)),
                    pl.BlockSpec((h_dim, 2 * h_dim), lambda i: (0, 0)),
                    pl.BlockSpec((1, 2 * h_dim), lambda i: (0, 0)),
                ],
                out_specs=[
                    pl.BlockSpec((nb, h_dim), lambda i: (i, 0)),
                    pl.BlockSpec((_NC, nb, h_dim), lambda i: (0, i, 0)),
                    pl.BlockSpec((_NC, nb, h_dim), lambda i: (0, i, 0)),
                ],
                out_shape=[
                    jax.ShapeDtypeStruct((n, h_dim), f32),
                    jax.ShapeDtypeStruct((_NC, n, h_dim), f32),
                    jax.ShapeDtypeStruct((_NC, n, h_dim), f32),
                ],
            )(agg, h, params['ln_g'][l][None, :], params['ln_b'][l][None, :],
              wd[l + 1], wsr[l + 1], bd[l + 1])
        else:
            h = pl.pallas_call(
                _update_only_body,
                grid=(n_grid,),
                in_specs=[
                    pl.BlockSpec((_NC, nb, 128), lambda i: (0, i, 0)),
                    pl.BlockSpec((nb, h_dim), lambda i: (i, 0)),
                    pl.BlockSpec((1, h_dim), lambda i: (0, 0)),
                    pl.BlockSpec((1, h_dim), lambda i: (0, 0)),
                ],
                out_specs=pl.BlockSpec((nb, h_dim), lambda i: (i, 0)),
                out_shape=jax.ShapeDtypeStruct((n, h_dim), f32),
            )(agg, h, params['ln_g'][l][None, :], params['ln_b'][l][None, :])

    out = pl.pallas_call(
        functools.partial(_pool_body, nblk=n_grid, blk=nb, g=g),
        grid=(n_grid,),
        in_specs=[
            pl.BlockSpec((nb, 1), lambda i: (i, 0)),
            pl.BlockSpec((nb, h_dim), lambda i: (i, 0)),
            pl.BlockSpec((h_dim, out_dim), lambda i: (0, 0)),
            pl.BlockSpec((1, out_dim), lambda i: (0, 0)),
        ],
        out_specs=pl.BlockSpec((g, out_dim), lambda i: (0, 0)),
        out_shape=jax.ShapeDtypeStruct((g, out_dim), f32),
        scratch_shapes=[
            pltpu.VMEM((g, h_dim), f32),
            pltpu.VMEM((g, h_dim), f32),
        ],
    )(batch[:, None], h, params['Wo'], params['bo'][None, :])

    return out


# speedup vs baseline: 2.3113x; 2.3113x over previous
import functools

import jax
import jax.numpy as jnp
from jax import lax
from jax.experimental import pallas as pl
from jax.experimental.pallas import tpu as pltpu
from jax.experimental.pallas import tpu_sc as plsc

_NC = 2
_NS = 16
_LN = 16
_CH = 24
_NEG = -1.0e9


def _silu(v):
    return v * (1.0 / (1.0 + jnp.exp(-v)))


def _h0_tables_body(x_ref, wi_ref, bi_ref, wd_ref, ws_ref, bd_ref,
                    h_ref, td_ref, ts_ref):
    a = jnp.dot(x_ref[...], wi_ref[...], preferred_element_type=jnp.float32)
    h = _silu(a + bi_ref[...])
    h_ref[...] = h
    d = jnp.dot(h, wd_ref[...], preferred_element_type=jnp.float32) + bd_ref[...]
    s = jnp.dot(h, ws_ref[...], preferred_element_type=jnp.float32)
    td_ref[0] = d[:, :256]
    td_ref[1] = d[:, 256:]
    ts_ref[0] = s[:, :256]
    ts_ref[1] = s[:, 256:]


def _update_tables_body(agg_ref, h_ref, g_ref, b_ref, wd_ref, ws_ref, bd_ref,
                        h_ref_out, td_ref, ts_ref):
    h = h_ref[...]
    agg = jnp.concatenate([agg_ref[0], agg_ref[1]], axis=-1)
    hn = _silu(agg + h) + h
    mu = jnp.mean(hn, axis=-1, keepdims=True)
    dvar = hn - mu
    var = jnp.mean(dvar * dvar, axis=-1, keepdims=True)
    hh = dvar * jax.lax.rsqrt(var + 1e-5) * g_ref[...] + b_ref[...]
    h_ref_out[...] = hh
    d = jnp.dot(hh, wd_ref[...], preferred_element_type=jnp.float32) + bd_ref[...]
    s = jnp.dot(hh, ws_ref[...], preferred_element_type=jnp.float32)
    td_ref[0] = d[:, :256]
    td_ref[1] = d[:, 256:]
    ts_ref[0] = s[:, :256]
    ts_ref[1] = s[:, 256:]


def _update_only_body(agg_ref, h_ref, g_ref, b_ref, h_ref_out):
    h = h_ref[...]
    agg = jnp.concatenate([agg_ref[0], agg_ref[1]], axis=-1)
    hn = _silu(agg + h) + h
    mu = jnp.mean(hn, axis=-1, keepdims=True)
    dvar = hn - mu
    var = jnp.mean(dvar * dvar, axis=-1, keepdims=True)
    h_ref_out[...] = dvar * jax.lax.rsqrt(var + 1e-5) * g_ref[...] + b_ref[...]


def _edgeproj_body(ea_ref, wa_ref, cc_ref, *, blk, e_real):
    i = pl.program_id(0)
    c = jnp.dot(ea_ref[...], wa_ref[...], preferred_element_type=jnp.float32)
    row = i * blk + lax.broadcasted_iota(jnp.int32, (blk, 1), 0)
    c = jnp.where(row < e_real, c, _NEG)
    cc_ref[0] = c[:, :256]
    cc_ref[1] = c[:, 256:]


def _pool_body(batch_ref, h_ref, wo_ref, bo_ref, out_ref, acc, cnt, *, nblk, blk, g):
    i = pl.program_id(0)

    @pl.when(i == 0)
    def _():
        acc[...] = jnp.zeros_like(acc)
        cnt[...] = jnp.zeros_like(cnt)

    ids = batch_ref[...]
    gids = lax.broadcasted_iota(jnp.int32, (blk, g), 1)
    oh = (ids == gids).astype(jnp.float32)
    h = h_ref[...]
    acc[...] += lax.dot_general(oh, h, (((0,), (0,)), ((), ())),
                                preferred_element_type=jnp.float32)
    cnt[...] += lax.dot_general(oh, jnp.ones_like(h), (((0,), (0,)), ((), ())),
                                preferred_element_type=jnp.float32)

    @pl.when(i == nblk - 1)
    def _():
        pooled = acc[...] / jnp.maximum(cnt[...], 1.0)
        out_ref[...] = (jnp.dot(pooled, wo_ref[...],
                                preferred_element_type=jnp.float32) + bo_ref[...])


def _edge_sc_body(td_hbm, ts_hbm, cc_hbm, dst_hbm, src_hbm, out_hbm,
                  dvec, svec, td_buf, ts_buf, cc_buf, m_buf, agg_sh,
                  sem_td, sem_ts, sem_cc, *, n_pad, ep_tile, nch):
    c = lax.axis_index("c")
    s = lax.axis_index("s")

    zero = jnp.zeros((_LN,), jnp.float32)

    def zrow(r, _):
        for j in range(128 // _LN):
            m_buf[r, pl.ds(j * _LN, _LN)] = zero
        return 0

    lax.fori_loop(0, _CH, zrow, 0)

    rpt = n_pad // _NS
    base = s * rpt
    off = 0
    while off < rpt:
        sz = min(_CH, rpt - off)
        pltpu.sync_copy(m_buf.at[pl.ds(0, sz)], agg_sh.at[pl.ds(base + off, sz)])
        off += sz
    plsc.subcore_barrier()

    def fetch(t, b):
        e0 = s * ep_tile + t * _CH
        pltpu.sync_copy(dst_hbm.at[pl.ds(e0, _CH)], dvec[b])
        pltpu.sync_copy(src_hbm.at[pl.ds(e0, _CH)], svec[b])
        pltpu.async_copy(td_hbm.at[c].at[dvec[b]], td_buf[b], sem_td[b])
        pltpu.async_copy(ts_hbm.at[c].at[svec[b]], ts_buf[b], sem_ts[b])
        pltpu.async_copy(cc_hbm.at[c].at[pl.ds(e0, _CH)], cc_buf[b], sem_cc[b])

    def process(t, b):
        e0 = s * ep_tile + t * _CH
        pltpu.make_async_copy(td_hbm.at[c].at[dvec[b]], td_buf[b], sem_td[b]).wait()
        pltpu.make_async_copy(ts_hbm.at[c].at[svec[b]], ts_buf[b], sem_ts[b]).wait()
        pltpu.make_async_copy(cc_hbm.at[c].at[pl.ds(e0, _CH)], cc_buf[b],
                              sem_cc[b]).wait()

        @plsc.parallel_loop(0, _CH, unroll=2)
        def edge(e):
            for j in range(8):
                jc = j * _LN
                af = td_buf[b][e, pl.ds(jc, _LN)] + ts_buf[b][e, pl.ds(jc, _LN)] \
                    + cc_buf[b][e, pl.ds(jc, _LN)]
                as_ = td_buf[b][e, pl.ds(128 + jc, _LN)] \
                    + ts_buf[b][e, pl.ds(128 + jc, _LN)] \
                    + cc_buf[b][e, pl.ds(128 + jc, _LN)]
                u = jnp.exp(-jnp.abs(as_))
                tt = u / (2.0 + u)
                t2 = tt * tt
                p = tt * (2.0 + t2 * (0.66666667 + t2 * (0.4 + t2 *
                          (0.28571429 + t2 * 0.22222222))))
                sp = jnp.maximum(as_, 0.0) + p
                m_buf[e, pl.ds(jc, _LN)] = sp / (1.0 + jnp.exp(-af))

        pltpu.sync_copy(m_buf, agg_sh.at[dvec[b]], add=True)

    fetch(0, 0)

    def chunk2(t2, _):
        t = t2 * 2
        fetch(t + 1, 1)
        process(t, 0)

        @pl.when(t2 < (nch // 2) - 1)
        def _():
            fetch(t + 2, 0)

        process(t + 1, 1)
        return 0

    lax.fori_loop(0, nch // 2, chunk2, 0)
    plsc.subcore_barrier()

    off = 0
    while off < rpt:
        sz = min(_CH, rpt - off)
        pltpu.sync_copy(agg_sh.at[pl.ds(base + off, sz)],
                        out_hbm.at[c].at[pl.ds(base + off, sz)])
        off += sz


def _edge_sc_call(td, ts, cc, dst_p, src_p, n_nodes, ep):
    ep_tile = ep // _NS
    nch = ep_tile // _CH
    n_pad = -(-n_nodes // (8 * _NS)) * (8 * _NS)
    mesh = plsc.VectorSubcoreMesh(core_axis_name="c", subcore_axis_name="s")
    body = functools.partial(_edge_sc_body, n_pad=n_pad,
                             ep_tile=ep_tile, nch=nch)

    def two(mk):
        return (mk(), mk())

    agg = pl.kernel(
        body,
        out_type=jax.ShapeDtypeStruct((_NC, n_pad, 128), jnp.float32),
        mesh=mesh,
        scratch_types=[
            two(lambda: pltpu.VMEM((_CH,), jnp.int32)),
            two(lambda: pltpu.VMEM((_CH,), jnp.int32)),
            two(lambda: pltpu.VMEM((_CH, 256), jnp.float32)),
            two(lambda: pltpu.VMEM((_CH, 256), jnp.float32)),
            two(lambda: pltpu.VMEM((_CH, 256), jnp.float32)),
            pltpu.VMEM((_CH, 128), jnp.float32),
            pltpu.VMEM_SHARED((n_pad, 128), jnp.float32),
            two(lambda: pltpu.SemaphoreType.DMA),
            two(lambda: pltpu.SemaphoreType.DMA),
            two(lambda: pltpu.SemaphoreType.DMA),
        ],
        name="edge_sc",
    )(td, ts, cc, dst_p, src_p)
    return agg[:, :n_nodes]


def _perm_w(wf, ws):
    return jnp.concatenate(
        [wf[:, :128], ws[:, :128], wf[:, 128:], ws[:, 128:]], axis=1)


def _perm_b(bf, bs):
    return jnp.concatenate([bf[:128], bs[:128], bf[128:], bs[128:]])[None, :]


def kernel(x, edge_attr, params, edge_index, batch):
    n, dd = x.shape
    e = edge_index.shape[1]
    h_dim = params['Wi'].shape[1]
    out_dim = params['Wo'].shape[1]
    n_layers = len(params['Wf'])
    g = 64

    q = _NS * _CH * 2
    ep = ((e + q - 1) // q) * q
    dst_p = jnp.concatenate(
        [edge_index[1], jnp.zeros((ep - e,), jnp.int32)]).astype(jnp.int32)
    src_p = jnp.concatenate(
        [edge_index[0], jnp.zeros((ep - e,), jnp.int32)]).astype(jnp.int32)
    ea_p = jnp.concatenate(
        [edge_attr, jnp.zeros((ep - e, edge_attr.shape[1]), jnp.float32)])

    wd = [_perm_w(params['Wf'][l][:h_dim], params['Ws'][l][:h_dim])
          for l in range(n_layers)]
    wsr = [_perm_w(params['Wf'][l][h_dim:2 * h_dim], params['Ws'][l][h_dim:2 * h_dim])
           for l in range(n_layers)]
    bd = [_perm_b(params['bf'][l], params['bs'][l]) for l in range(n_layers)]
    wa = [_perm_w(params['Wf'][l][2 * h_dim:], params['Ws'][l][2 * h_dim:])
          for l in range(n_layers)]

    nb = 1000
    n_grid = n // nb
    eb = next(b for b in (2048, 2000, 1600, 1024, 768, 512, 256, 128)
              if ep % b == 0)
    e_grid = ep // eb

    f32 = jnp.float32

    h, td, ts = pl.pallas_call(
        _h0_tables_body,
        grid=(n_grid,),
        in_specs=[
            pl.BlockSpec((nb, dd), lambda i: (i, 0)),
            pl.BlockSpec((dd, h_dim), lambda i: (0, 0)),
            pl.BlockSpec((1, h_dim), lambda i: (0, 0)),
            pl.BlockSpec((h_dim, 2 * h_dim), lambda i: (0, 0)),
            pl.BlockSpec((h_dim, 2 * h_dim), lambda i: (0, 0)),
            pl.BlockSpec((1, 2 * h_dim), lambda i: (0, 0)),
        ],
        out_specs=[
            pl.BlockSpec((nb, h_dim), lambda i: (i, 0)),
            pl.BlockSpec((_NC, nb, h_dim), lambda i: (0, i, 0)),
            pl.BlockSpec((_NC, nb, h_dim), lambda i: (0, i, 0)),
        ],
        out_shape=[
            jax.ShapeDtypeStruct((n, h_dim), f32),
            jax.ShapeDtypeStruct((_NC, n, h_dim), f32),
            jax.ShapeDtypeStruct((_NC, n, h_dim), f32),
        ],
    )(x, params['Wi'], params['bi'][None, :], wd[0], wsr[0], bd[0])

    for l in range(n_layers):
        cc = pl.pallas_call(
            functools.partial(_edgeproj_body, blk=eb, e_real=e),
            grid=(e_grid,),
            in_specs=[
                pl.BlockSpec((eb, ea_p.shape[1]), lambda i: (i, 0)),
                pl.BlockSpec((ea_p.shape[1], 2 * h_dim), lambda i: (0, 0)),
            ],
            out_specs=pl.BlockSpec((_NC, eb, h_dim), lambda i: (0, i, 0)),
            out_shape=jax.ShapeDtypeStruct((_NC, ep, h_dim), f32),
        )(ea_p, wa[l])

        agg = _edge_sc_call(td, ts, cc, dst_p, src_p, n, ep)

        if l + 1 < n_layers:
            h, td, ts = pl.pallas_call(
                _update_tables_body,
                grid=(n_grid,),
                in_specs=[
                    pl.BlockSpec((_NC, nb, 128), lambda i: (0, i, 0)),
                    pl.BlockSpec((nb, h_dim), lambda i: (i, 0)),
                    pl.BlockSpec((1, h_dim), lambda i: (0, 0)),
                    pl.BlockSpec((1, h_dim), lambda i: (0, 0)),
                    pl.BlockSpec((h_dim, 2 * h_dim), lambda i: (0, 0)),
                    pl.BlockSpec((h_dim, 2 * h_dim), lambda i: (0, 0)),
                    pl.BlockSpec((1, 2 * h_dim), lambda i: (0, 0)),
                ],
                out_specs=[
                    pl.BlockSpec((nb, h_dim), lambda i: (i, 0)),
                    pl.BlockSpec((_NC, nb, h_dim), lambda i: (0, i, 0)),
                    pl.BlockSpec((_NC, nb, h_dim), lambda i: (0, i, 0)),
                ],
                out_shape=[
                    jax.ShapeDtypeStruct((n, h_dim), f32),
                    jax.ShapeDtypeStruct((_NC, n, h_dim), f32),
                    jax.ShapeDtypeStruct((_NC, n, h_dim), f32),
                ],
            )(agg, h, params['ln_g'][l][None, :], params['ln_b'][l][None, :],
              wd[l + 1], wsr[l + 1], bd[l + 1])
        else:
            h = pl.pallas_call(
                _update_only_body,
                grid=(n_grid,),
                in_specs=[
                    pl.BlockSpec((_NC, nb, 128), lambda i: (0, i, 0)),
                    pl.BlockSpec((nb, h_dim), lambda i: (i, 0)),
                    pl.BlockSpec((1, h_dim), lambda i: (0, 0)),
                    pl.BlockSpec((1, h_dim), lambda i: (0, 0)),
                ],
                out_specs=pl.BlockSpec((nb, h_dim), lambda i: (i, 0)),
                out_shape=jax.ShapeDtypeStruct((n, h_dim), f32),
            )(agg, h, params['ln_g'][l][None, :], params['ln_b'][l][None, :])

    out = pl.pallas_call(
        functools.partial(_pool_body, nblk=n_grid, blk=nb, g=g),
        grid=(n_grid,),
        in_specs=[
            pl.BlockSpec((nb, 1), lambda i: (i, 0)),
            pl.BlockSpec((nb, h_dim), lambda i: (i, 0)),
            pl.BlockSpec((h_dim, out_dim), lambda i: (0, 0)),
            pl.BlockSpec((1, out_dim), lambda i: (0, 0)),
        ],
        out_specs=pl.BlockSpec((g, out_dim), lambda i: (0, 0)),
        out_shape=jax.ShapeDtypeStruct((g, out_dim), f32),
        scratch_shapes=[
            pltpu.VMEM((g, h_dim), f32),
            pltpu.VMEM((g, h_dim), f32),
        ],
    )(batch[:, None], h, params['Wo'], params['bo'][None, :])

    return out


# speedup vs baseline: 5.0818x; 2.1987x over previous
import functools

import jax
import jax.numpy as jnp
from jax import lax
from jax.experimental import pallas as pl
from jax.experimental.pallas import tpu as pltpu
from jax.experimental.pallas import tpu_sc as plsc

_NC = 2
_NS = 16
_LN = 16
_CHA = 64
_NBUF = 4
_CHC = 128
_SB = 8


def _silu(v):
    return v * (1.0 / (1.0 + jnp.exp(-v)))


def _pack_tables(d, ref):
    for c in range(_NC):
        gb = lax.bitcast_convert_type(d[:, 256 * c:256 * c + 128], jnp.int32)
        sb = lax.bitcast_convert_type(d[:, 256 * c + 128:256 * c + 256],
                                      jnp.int32)
        lo = ((gb + 0x8000) >> 16) & 0xFFFF
        hi = (sb + 0x8000) & (-65536)
        ref[c] = lo | hi


def _unpack_lo(w):
    return lax.bitcast_convert_type(w << 16, jnp.float32)


def _unpack_hi(w):
    return lax.bitcast_convert_type(w & (-65536), jnp.float32)


def _h0_tables_body(x_ref, wi_ref, bi_ref, wd_ref, ws_ref, bd_ref,
                    h_ref, td_ref, ts_ref):
    a = jnp.dot(x_ref[...], wi_ref[...], preferred_element_type=jnp.float32)
    h = _silu(a + bi_ref[...])
    h_ref[...] = h
    d = jnp.dot(h, wd_ref[...], preferred_element_type=jnp.float32) + bd_ref[...]
    s = jnp.dot(h, ws_ref[...], preferred_element_type=jnp.float32)
    _pack_tables(d, td_ref)
    _pack_tables(s, ts_ref)


def _update_tables_body(agg_ref, h_ref, g_ref, b_ref, wd_ref, ws_ref, bd_ref,
                        h_ref_out, td_ref, ts_ref):
    h = h_ref[...]
    agg = jnp.concatenate([agg_ref[0], agg_ref[1]], axis=-1)
    hn = _silu(agg + h) + h
    mu = jnp.mean(hn, axis=-1, keepdims=True)
    dvar = hn - mu
    var = jnp.mean(dvar * dvar, axis=-1, keepdims=True)
    hh = dvar * jax.lax.rsqrt(var + 1e-5) * g_ref[...] + b_ref[...]
    h_ref_out[...] = hh
    d = jnp.dot(hh, wd_ref[...], preferred_element_type=jnp.float32) + bd_ref[...]
    s = jnp.dot(hh, ws_ref[...], preferred_element_type=jnp.float32)
    _pack_tables(d, td_ref)
    _pack_tables(s, ts_ref)


def _update_only_body(agg_ref, h_ref, g_ref, b_ref, h_ref_out):
    h = h_ref[...]
    agg = jnp.concatenate([agg_ref[0], agg_ref[1]], axis=-1)
    hn = _silu(agg + h) + h
    mu = jnp.mean(hn, axis=-1, keepdims=True)
    dvar = hn - mu
    var = jnp.mean(dvar * dvar, axis=-1, keepdims=True)
    h_ref_out[...] = dvar * jax.lax.rsqrt(var + 1e-5) * g_ref[...] + b_ref[...]


def _msg_body(sd_ref, ss_ref, ea_ref, wa_ref, m_ref, *, blk, e_real):
    i = pl.program_id(0)
    cc = jnp.dot(ea_ref[...], wa_ref[...], preferred_element_type=jnp.float32)
    row = i * blk + lax.broadcasted_iota(jnp.int32, (blk, 1), 0)
    live = row < e_real
    for c in range(_NC):
        wd = sd_ref[c]
        ws = ss_ref[c]
        f = _unpack_lo(wd) + _unpack_lo(ws) + cc[:, 256 * c:256 * c + 128]
        s = _unpack_hi(wd) + _unpack_hi(ws) + cc[:, 256 * c + 128:256 * c + 256]
        sp = jnp.maximum(s, 0.0) + jnp.log1p(jnp.exp(-jnp.abs(s)))
        m = sp / (1.0 + jnp.exp(-f))
        m_ref[c] = jnp.where(live, m, 0.0)


def _pool_body(batch_ref, h_ref, wo_ref, bo_ref, out_ref, acc, cnt, *, nblk, blk, g):
    i = pl.program_id(0)

    @pl.when(i == 0)
    def _():
        acc[...] = jnp.zeros_like(acc)
        cnt[...] = jnp.zeros_like(cnt)

    ids = batch_ref[...]
    gids = lax.broadcasted_iota(jnp.int32, (blk, g), 1)
    oh = (ids == gids).astype(jnp.float32)
    h = h_ref[...]
    acc[...] += lax.dot_general(oh, h, (((0,), (0,)), ((), ())),
                                preferred_element_type=jnp.float32)
    cnt[...] += lax.dot_general(oh, jnp.ones_like(h), (((0,), (0,)), ((), ())),
                                preferred_element_type=jnp.float32)

    @pl.when(i == nblk - 1)
    def _():
        pooled = acc[...] / jnp.maximum(cnt[...], 1.0)
        out_ref[...] = (jnp.dot(pooled, wo_ref[...],
                                preferred_element_type=jnp.float32) + bo_ref[...])


def _gather_sc_body(td_hbm, ts_hbm, dst_hbm, src_hbm, sd_hbm, ss_hbm,
                    didx_d, didx_s, dvec, svec, dbuf, sbuf,
                    sem_gd, sem_gs, sem_od, sem_os, *, ep_tile, nch):
    c = lax.axis_index("c")
    s = lax.axis_index("s")

    def fetch(t, b):
        @pl.when((t % _SB) == 0)
        def _():
            r0 = pl.multiple_of(s * nch + t, _SB)
            pltpu.sync_copy(dst_hbm.at[pl.ds(r0, _SB)], didx_d)
            pltpu.sync_copy(src_hbm.at[pl.ds(r0, _SB)], didx_s)

        k = t % _SB
        for hh in range(_CHA // _LN):
            hc = hh * _LN
            dvec[b][pl.ds(hc, _LN)] = didx_d[k, pl.ds(hc, _LN)]
            svec[b][pl.ds(hc, _LN)] = didx_s[k, pl.ds(hc, _LN)]
        pltpu.async_copy(td_hbm.at[c].at[dvec[b]], dbuf[b], sem_gd[b])
        pltpu.async_copy(ts_hbm.at[c].at[svec[b]], sbuf[b], sem_gs[b])

    def flush(t, b):
        e0 = s * ep_tile + t * _CHA
        pltpu.make_async_copy(td_hbm.at[c].at[dvec[b]], dbuf[b], sem_gd[b]).wait()
        pltpu.make_async_copy(ts_hbm.at[c].at[svec[b]], sbuf[b], sem_gs[b]).wait()
        pltpu.async_copy(dbuf[b], sd_hbm.at[c].at[pl.ds(e0, _CHA)], sem_od[b])
        pltpu.async_copy(sbuf[b], ss_hbm.at[c].at[pl.ds(e0, _CHA)], sem_os[b])

    def wait_out(t, b):
        e0 = s * ep_tile + t * _CHA
        pltpu.make_async_copy(dbuf[b], sd_hbm.at[c].at[pl.ds(e0, _CHA)],
                              sem_od[b]).wait()
        pltpu.make_async_copy(sbuf[b], ss_hbm.at[c].at[pl.ds(e0, _CHA)],
                              sem_os[b]).wait()

    for j in range(_NBUF):
        fetch(j, j)

    nq = nch // _NBUF

    def quad(q, _):
        t = q * _NBUF
        for j in range(_NBUF):
            flush(t + j, j)

        @pl.when(q < nq - 1)
        def _():
            for j in range(_NBUF):
                wait_out(t + j, j)
                fetch(t + _NBUF + j, j)

        return 0

    lax.fori_loop(0, nq, quad, 0)
    for j in range(_NBUF):
        wait_out((nq - 1) * _NBUF + j, j)


def _scatter_sc_body(m_hbm, dst_hbm, out_hbm,
                     didx, dvec, m_buf, agg_sh, sem_m,
                     *, n_pad, ep_tile, nch):
    c = lax.axis_index("c")
    s = lax.axis_index("s")

    zero = jnp.zeros((_LN,), jnp.float32)

    def zrow(r, _):
        for j in range(128 // _LN):
            m_buf[0][r, pl.ds(j * _LN, _LN)] = zero
        return 0

    lax.fori_loop(0, _CHC, zrow, 0)
    rpt = n_pad // _NS
    base = s * rpt
    off = 0
    while off < rpt:
        sz = min(_CHC, rpt - off)
        pltpu.sync_copy(m_buf[0].at[pl.ds(0, sz)],
                        agg_sh.at[pl.ds(base + off, sz)])
        off += sz
    plsc.subcore_barrier()

    def fetch(t, b):
        @pl.when((t % _SB) == 0)
        def _():
            r0 = pl.multiple_of(s * nch + t, _SB)
            pltpu.sync_copy(dst_hbm.at[pl.ds(r0, _SB)], didx)

        k = t % _SB
        for hh in range(_CHC // _LN):
            hc = hh * _LN
            dvec[b][pl.ds(hc, _LN)] = didx[k, pl.ds(hc, _LN)]
        e0 = s * ep_tile + t * _CHC
        pltpu.async_copy(m_hbm.at[c].at[pl.ds(e0, _CHC)], m_buf[b], sem_m[b])

    def process(t, b):
        e0 = s * ep_tile + t * _CHC
        pltpu.make_async_copy(m_hbm.at[c].at[pl.ds(e0, _CHC)], m_buf[b],
                              sem_m[b]).wait()
        pltpu.sync_copy(m_buf[b], agg_sh.at[dvec[b]], add=True)

    fetch(0, 0)

    def pair(t2, _):
        t = t2 * 2
        fetch(t + 1, 1)
        process(t, 0)

        @pl.when(t2 < (nch // 2) - 1)
        def _():
            fetch(t + 2, 0)

        process(t + 1, 1)
        return 0

    lax.fori_loop(0, nch // 2, pair, 0)
    plsc.subcore_barrier()

    off = 0
    while off < rpt:
        sz = min(_CHC, rpt - off)
        pltpu.sync_copy(agg_sh.at[pl.ds(base + off, sz)],
                        out_hbm.at[c].at[pl.ds(base + off, sz)])
        off += sz


def _gather_sc_call(td, ts, dst2, src2, ep):
    ep_tile = ep // _NS
    nch = ep_tile // _CHA
    mesh = plsc.VectorSubcoreMesh(core_axis_name="c", subcore_axis_name="s")
    body = functools.partial(_gather_sc_body, ep_tile=ep_tile, nch=nch)

    def nbuf(mk):
        return tuple(mk() for _ in range(_NBUF))

    i32 = jnp.int32
    sd, ss = pl.kernel(
        body,
        out_type=[jax.ShapeDtypeStruct((_NC, ep, 128), i32),
                  jax.ShapeDtypeStruct((_NC, ep, 128), i32)],
        mesh=mesh,
        scratch_types=[
            pltpu.VMEM((_SB, _CHA), i32),
            pltpu.VMEM((_SB, _CHA), i32),
            nbuf(lambda: pltpu.VMEM((_CHA,), i32)),
            nbuf(lambda: pltpu.VMEM((_CHA,), i32)),
            nbuf(lambda: pltpu.VMEM((_CHA, 128), i32)),
            nbuf(lambda: pltpu.VMEM((_CHA, 128), i32)),
            nbuf(lambda: pltpu.SemaphoreType.DMA),
            nbuf(lambda: pltpu.SemaphoreType.DMA),
            nbuf(lambda: pltpu.SemaphoreType.DMA),
            nbuf(lambda: pltpu.SemaphoreType.DMA),
        ],
        name="gather_sc",
    )(td, ts, dst2, src2)
    return sd, ss


def _scatter_sc_call(m, dst2, n_nodes, ep):
    ep_tile = ep // _NS
    nch = ep_tile // _CHC
    n_pad = -(-n_nodes // (8 * _NS)) * (8 * _NS)
    mesh = plsc.VectorSubcoreMesh(core_axis_name="c", subcore_axis_name="s")
    body = functools.partial(_scatter_sc_body, n_pad=n_pad,
                             ep_tile=ep_tile, nch=nch)

    def two(mk):
        return (mk(), mk())

    agg = pl.kernel(
        body,
        out_type=jax.ShapeDtypeStruct((_NC, n_pad, 128), jnp.float32),
        mesh=mesh,
        scratch_types=[
            pltpu.VMEM((_SB, _CHC), jnp.int32),
            two(lambda: pltpu.VMEM((_CHC,), jnp.int32)),
            two(lambda: pltpu.VMEM((_CHC, 128), jnp.float32)),
            pltpu.VMEM_SHARED((n_pad, 128), jnp.float32),
            two(lambda: pltpu.SemaphoreType.DMA),
        ],
        name="scatter_sc",
    )(m, dst2)
    return agg[:, :n_nodes]


def _perm_w(wf, ws):
    return jnp.concatenate(
        [wf[:, :128], ws[:, :128], wf[:, 128:], ws[:, 128:]], axis=1)


def _perm_b(bf, bs):
    return jnp.concatenate([bf[:128], bs[:128], bf[128:], bs[128:]])[None, :]


def kernel(x, edge_attr, params, edge_index, batch):
    n, dd = x.shape
    e = edge_index.shape[1]
    h_dim = params['Wi'].shape[1]
    out_dim = params['Wo'].shape[1]
    n_layers = len(params['Wf'])
    g = 64

    q = _NS * _SB * max(_CHA * _NBUF, _CHC * 2)
    ep = ((e + q - 1) // q) * q
    dst_p = jnp.concatenate(
        [edge_index[1], jnp.zeros((ep - e,), jnp.int32)]).astype(jnp.int32)
    src_p = jnp.concatenate(
        [edge_index[0], jnp.zeros((ep - e,), jnp.int32)]).astype(jnp.int32)
    dst_a = dst_p.reshape(-1, _CHA)
    src_a = src_p.reshape(-1, _CHA)
    dst_c = dst_p.reshape(-1, _CHC)
    ea_p = jnp.concatenate(
        [edge_attr, jnp.zeros((ep - e, edge_attr.shape[1]), jnp.float32)])

    wd = [_perm_w(params['Wf'][l][:h_dim], params['Ws'][l][:h_dim])
          for l in range(n_layers)]
    wsr = [_perm_w(params['Wf'][l][h_dim:2 * h_dim], params['Ws'][l][h_dim:2 * h_dim])
           for l in range(n_layers)]
    bd = [_perm_b(params['bf'][l], params['bs'][l]) for l in range(n_layers)]
    wa = [_perm_w(params['Wf'][l][2 * h_dim:], params['Ws'][l][2 * h_dim:])
          for l in range(n_layers)]

    nb = 1000
    n_grid = n // nb
    eb = 2048
    e_grid = ep // eb

    f32 = jnp.float32

    h, td, ts = pl.pallas_call(
        _h0_tables_body,
        grid=(n_grid,),
        in_specs=[
            pl.BlockSpec((nb, dd), lambda i: (i, 0)),
            pl.BlockSpec((dd, h_dim), lambda i: (0, 0)),
            pl.BlockSpec((1, h_dim), lambda i: (0, 0)),
            pl.BlockSpec((h_dim, 2 * h_dim), lambda i: (0, 0)),
            pl.BlockSpec((h_dim, 2 * h_dim), lambda i: (0, 0)),
            pl.BlockSpec((1, 2 * h_dim), lambda i: (0, 0)),
        ],
        out_specs=[
            pl.BlockSpec((nb, h_dim), lambda i: (i, 0)),
            pl.BlockSpec((_NC, nb, 128), lambda i: (0, i, 0)),
            pl.BlockSpec((_NC, nb, 128), lambda i: (0, i, 0)),
        ],
        out_shape=[
            jax.ShapeDtypeStruct((n, h_dim), f32),
            jax.ShapeDtypeStruct((_NC, n, 128), jnp.int32),
            jax.ShapeDtypeStruct((_NC, n, 128), jnp.int32),
        ],
    )(x, params['Wi'], params['bi'][None, :], wd[0], wsr[0], bd[0])

    for l in range(n_layers):
        sd, ss = _gather_sc_call(td, ts, dst_a, src_a, ep)

        m = pl.pallas_call(
            functools.partial(_msg_body, blk=eb, e_real=e),
            grid=(e_grid,),
            in_specs=[
                pl.BlockSpec((_NC, eb, 128), lambda i: (0, i, 0)),
                pl.BlockSpec((_NC, eb, 128), lambda i: (0, i, 0)),
                pl.BlockSpec((eb, ea_p.shape[1]), lambda i: (i, 0)),
                pl.BlockSpec((ea_p.shape[1], 2 * h_dim), lambda i: (0, 0)),
            ],
            out_specs=pl.BlockSpec((_NC, eb, 128), lambda i: (0, i, 0)),
            out_shape=jax.ShapeDtypeStruct((_NC, ep, 128), f32),
        )(sd, ss, ea_p, wa[l])

        agg = _scatter_sc_call(m, dst_c, n, ep)

        if l + 1 < n_layers:
            h, td, ts = pl.pallas_call(
                _update_tables_body,
                grid=(n_grid,),
                in_specs=[
                    pl.BlockSpec((_NC, nb, 128), lambda i: (0, i, 0)),
                    pl.BlockSpec((nb, h_dim), lambda i: (i, 0)),
                    pl.BlockSpec((1, h_dim), lambda i: (0, 0)),
                    pl.BlockSpec((1, h_dim), lambda i: (0, 0)),
                    pl.BlockSpec((h_dim, 2 * h_dim), lambda i: (0, 0)),
                    pl.BlockSpec((h_dim, 2 * h_dim), lambda i: (0, 0)),
                    pl.BlockSpec((1, 2 * h_dim), lambda i: (0, 0)),
                ],
                out_specs=[
                    pl.BlockSpec((nb, h_dim), lambda i: (i, 0)),
                    pl.BlockSpec((_NC, nb, 128), lambda i: (0, i, 0)),
                    pl.BlockSpec((_NC, nb, 128), lambda i: (0, i, 0)),
                ],
                out_shape=[
                    jax.ShapeDtypeStruct((n, h_dim), f32),
                    jax.ShapeDtypeStruct((_NC, n, 128), jnp.int32),
                    jax.ShapeDtypeStruct((_NC, n, 128), jnp.int32),
                ],
            )(agg, h, params['ln_g'][l][None, :], params['ln_b'][l][None, :],
              wd[l + 1], wsr[l + 1], bd[l + 1])
        else:
            h = pl.pallas_call(
                _update_only_body,
                grid=(n_grid,),
                in_specs=[
                    pl.BlockSpec((_NC, nb, 128), lambda i: (0, i, 0)),
                    pl.BlockSpec((nb, h_dim), lambda i: (i, 0)),
                    pl.BlockSpec((1, h_dim), lambda i: (0, 0)),
                    pl.BlockSpec((1, h_dim), lambda i: (0, 0)),
                ],
                out_specs=pl.BlockSpec((nb, h_dim), lambda i: (i, 0)),
                out_shape=jax.ShapeDtypeStruct((n, h_dim), f32),
            )(agg, h, params['ln_g'][l][None, :], params['ln_b'][l][None, :])

    out = pl.pallas_call(
        functools.partial(_pool_body, nblk=n_grid, blk=nb, g=g),
        grid=(n_grid,),
        in_specs=[
            pl.BlockSpec((nb, 1), lambda i: (i, 0)),
            pl.BlockSpec((nb, h_dim), lambda i: (i, 0)),
            pl.BlockSpec((h_dim, out_dim), lambda i: (0, 0)),
            pl.BlockSpec((1, out_dim), lambda i: (0, 0)),
        ],
        out_specs=pl.BlockSpec((g, out_dim), lambda i: (0, 0)),
        out_shape=jax.ShapeDtypeStruct((g, out_dim), f32),
        scratch_shapes=[
            pltpu.VMEM((g, h_dim), f32),
            pltpu.VMEM((g, h_dim), f32),
        ],
    )(batch[:, None], h, params['Wo'], params['bo'][None, :])

    return out


# speedup vs baseline: 5.3803x; 1.0587x over previous
import functools

import jax
import jax.numpy as jnp
from jax import lax
from jax.experimental import pallas as pl
from jax.experimental.pallas import tpu as pltpu
from jax.experimental.pallas import tpu_sc as plsc

_NC = 2
_NS = 16
_LN = 16
_CHA = 64
_NBUF = 5
_PD = 3
_CHC = 128
_SB = 8


def _silu(v):
    return v * (1.0 / (1.0 + jnp.exp(-v)))


def _pack_tables(d, ref):
    for c in range(_NC):
        gb = lax.bitcast_convert_type(d[:, 256 * c:256 * c + 128], jnp.int32)
        sb = lax.bitcast_convert_type(d[:, 256 * c + 128:256 * c + 256],
                                      jnp.int32)
        lo = ((gb + 0x8000) >> 16) & 0xFFFF
        hi = (sb + 0x8000) & (-65536)
        ref[c] = lo | hi


def _unpack_lo(w):
    return lax.bitcast_convert_type(w << 16, jnp.float32)


def _unpack_hi(w):
    return lax.bitcast_convert_type(w & (-65536), jnp.float32)


def _h0_tables_body(x_ref, wi_ref, bi_ref, wd_ref, ws_ref, bd_ref,
                    h_ref, td_ref, ts_ref):
    a = jnp.dot(x_ref[...], wi_ref[...], preferred_element_type=jnp.float32)
    h = _silu(a + bi_ref[...])
    h_ref[...] = h
    d = jnp.dot(h, wd_ref[...], preferred_element_type=jnp.float32) + bd_ref[...]
    s = jnp.dot(h, ws_ref[...], preferred_element_type=jnp.float32)
    _pack_tables(d, td_ref)
    _pack_tables(s, ts_ref)


def _update_tables_body(agg_ref, h_ref, g_ref, b_ref, wd_ref, ws_ref, bd_ref,
                        h_ref_out, td_ref, ts_ref):
    h = h_ref[...]
    agg = jnp.concatenate([agg_ref[0], agg_ref[1]], axis=-1)
    hn = _silu(agg + h) + h
    mu = jnp.mean(hn, axis=-1, keepdims=True)
    dvar = hn - mu
    var = jnp.mean(dvar * dvar, axis=-1, keepdims=True)
    hh = dvar * jax.lax.rsqrt(var + 1e-5) * g_ref[...] + b_ref[...]
    h_ref_out[...] = hh
    d = jnp.dot(hh, wd_ref[...], preferred_element_type=jnp.float32) + bd_ref[...]
    s = jnp.dot(hh, ws_ref[...], preferred_element_type=jnp.float32)
    _pack_tables(d, td_ref)
    _pack_tables(s, ts_ref)


def _update_only_body(agg_ref, h_ref, g_ref, b_ref, h_ref_out):
    h = h_ref[...]
    agg = jnp.concatenate([agg_ref[0], agg_ref[1]], axis=-1)
    hn = _silu(agg + h) + h
    mu = jnp.mean(hn, axis=-1, keepdims=True)
    dvar = hn - mu
    var = jnp.mean(dvar * dvar, axis=-1, keepdims=True)
    h_ref_out[...] = dvar * jax.lax.rsqrt(var + 1e-5) * g_ref[...] + b_ref[...]


def _msg_body(sd_ref, ss_ref, ea_ref, wa_ref, m_ref, *, blk, e_real):
    i = pl.program_id(0)
    cc = jnp.dot(ea_ref[...], wa_ref[...], preferred_element_type=jnp.float32)
    row = i * blk + lax.broadcasted_iota(jnp.int32, (blk, 1), 0)
    live = row < e_real
    for c in range(_NC):
        wd = sd_ref[c]
        ws = ss_ref[c]
        f = _unpack_lo(wd) + _unpack_lo(ws) + cc[:, 256 * c:256 * c + 128]
        s = _unpack_hi(wd) + _unpack_hi(ws) + cc[:, 256 * c + 128:256 * c + 256]
        sp = jnp.maximum(s, 0.0) + jnp.log1p(jnp.exp(-jnp.abs(s)))
        m = sp / (1.0 + jnp.exp(-f))
        m_ref[c] = jnp.where(live, m, 0.0)


def _pool_body(batch_ref, h_ref, wo_ref, bo_ref, out_ref, acc, cnt, *, nblk, blk, g):
    i = pl.program_id(0)

    @pl.when(i == 0)
    def _():
        acc[...] = jnp.zeros_like(acc)
        cnt[...] = jnp.zeros_like(cnt)

    ids = batch_ref[...]
    gids = lax.broadcasted_iota(jnp.int32, (blk, g), 1)
    oh = (ids == gids).astype(jnp.float32)
    h = h_ref[...]
    acc[...] += lax.dot_general(oh, h, (((0,), (0,)), ((), ())),
                                preferred_element_type=jnp.float32)
    cnt[...] += lax.dot_general(oh, jnp.ones_like(h), (((0,), (0,)), ((), ())),
                                preferred_element_type=jnp.float32)

    @pl.when(i == nblk - 1)
    def _():
        pooled = acc[...] / jnp.maximum(cnt[...], 1.0)
        out_ref[...] = (jnp.dot(pooled, wo_ref[...],
                                preferred_element_type=jnp.float32) + bo_ref[...])


def _gather_sc_body(td_hbm, ts_hbm, dst_hbm, src_hbm, sd_hbm, ss_hbm,
                    didx_d, didx_s, dvec, svec, dbuf, sbuf,
                    sem_gd, sem_gs, sem_od, sem_os, *, ep_tile, nch):
    c = lax.axis_index("c")
    s = lax.axis_index("s")

    def fetch(t, b):
        @pl.when((t % _SB) == 0)
        def _():
            r0 = pl.multiple_of(s * nch + t, _SB)
            pltpu.sync_copy(dst_hbm.at[pl.ds(r0, _SB)], didx_d)
            pltpu.sync_copy(src_hbm.at[pl.ds(r0, _SB)], didx_s)

        k = t % _SB
        for hh in range(_CHA // _LN):
            hc = hh * _LN
            dvec[b][pl.ds(hc, _LN)] = didx_d[k, pl.ds(hc, _LN)]
            svec[b][pl.ds(hc, _LN)] = didx_s[k, pl.ds(hc, _LN)]
        pltpu.async_copy(td_hbm.at[c].at[dvec[b]], dbuf[b], sem_gd[b])
        pltpu.async_copy(ts_hbm.at[c].at[svec[b]], sbuf[b], sem_gs[b])

    def flush(t, b):
        e0 = s * ep_tile + t * _CHA
        pltpu.make_async_copy(td_hbm.at[c].at[dvec[b]], dbuf[b], sem_gd[b]).wait()
        pltpu.make_async_copy(ts_hbm.at[c].at[svec[b]], sbuf[b], sem_gs[b]).wait()
        pltpu.async_copy(dbuf[b], sd_hbm.at[c].at[pl.ds(e0, _CHA)], sem_od[b])
        pltpu.async_copy(sbuf[b], ss_hbm.at[c].at[pl.ds(e0, _CHA)], sem_os[b])

    def wait_out(t, b):
        e0 = s * ep_tile + t * _CHA
        pltpu.make_async_copy(dbuf[b], sd_hbm.at[c].at[pl.ds(e0, _CHA)],
                              sem_od[b]).wait()
        pltpu.make_async_copy(sbuf[b], ss_hbm.at[c].at[pl.ds(e0, _CHA)],
                              sem_os[b]).wait()

    for j in range(_PD):
        fetch(j, j)

    def group(gq, _):
        t0 = gq * _NBUF
        for j in range(_NBUF):
            t = t0 + j
            flush(t, j)
            tp = t + _PD
            bp = (j + _PD) % _NBUF

            @pl.when(tp - _NBUF >= 0)
            def _():
                wait_out(tp - _NBUF, bp)

            @pl.when(tp < nch)
            def _():
                fetch(tp, bp)

        return 0

    lax.fori_loop(0, nch // _NBUF, group, 0)
    for t in range(nch - (_NBUF - _PD), nch):
        wait_out(t, t % _NBUF)


def _scatter_sc_body(m_hbm, dst_hbm, out_hbm,
                     didx, dvec, m_buf, agg_sh, sem_m,
                     *, n_pad, ep_tile, nch):
    c = lax.axis_index("c")
    s = lax.axis_index("s")

    zero = jnp.zeros((_LN,), jnp.float32)

    def zrow(r, _):
        for j in range(128 // _LN):
            m_buf[0][r, pl.ds(j * _LN, _LN)] = zero
        return 0

    lax.fori_loop(0, _CHC, zrow, 0)
    rpt = n_pad // _NS
    base = s * rpt
    off = 0
    while off < rpt:
        sz = min(_CHC, rpt - off)
        pltpu.sync_copy(m_buf[0].at[pl.ds(0, sz)],
                        agg_sh.at[pl.ds(base + off, sz)])
        off += sz
    plsc.subcore_barrier()

    def fetch(t, b):
        @pl.when((t % _SB) == 0)
        def _():
            r0 = pl.multiple_of(s * nch + t, _SB)
            pltpu.sync_copy(dst_hbm.at[pl.ds(r0, _SB)], didx)

        k = t % _SB
        for hh in range(_CHC // _LN):
            hc = hh * _LN
            dvec[b][pl.ds(hc, _LN)] = didx[k, pl.ds(hc, _LN)]
        e0 = s * ep_tile + t * _CHC
        pltpu.async_copy(m_hbm.at[c].at[pl.ds(e0, _CHC)], m_buf[b], sem_m[b])

    def process(t, b):
        e0 = s * ep_tile + t * _CHC
        pltpu.make_async_copy(m_hbm.at[c].at[pl.ds(e0, _CHC)], m_buf[b],
                              sem_m[b]).wait()
        pltpu.sync_copy(m_buf[b], agg_sh.at[dvec[b]], add=True)

    fetch(0, 0)

    def pair(t2, _):
        t = t2 * 2
        fetch(t + 1, 1)
        process(t, 0)

        @pl.when(t2 < (nch // 2) - 1)
        def _():
            fetch(t + 2, 0)

        process(t + 1, 1)
        return 0

    lax.fori_loop(0, nch // 2, pair, 0)
    plsc.subcore_barrier()

    off = 0
    while off < rpt:
        sz = min(_CHC, rpt - off)
        pltpu.sync_copy(agg_sh.at[pl.ds(base + off, sz)],
                        out_hbm.at[c].at[pl.ds(base + off, sz)])
        off += sz


def _gather_sc_call(td, ts, dst2, src2, ep):
    ep_tile = ep // _NS
    nch = ep_tile // _CHA
    mesh = plsc.VectorSubcoreMesh(core_axis_name="c", subcore_axis_name="s")
    body = functools.partial(_gather_sc_body, ep_tile=ep_tile, nch=nch)

    def nbuf(mk):
        return tuple(mk() for _ in range(_NBUF))

    i32 = jnp.int32
    sd, ss = pl.kernel(
        body,
        out_type=[jax.ShapeDtypeStruct((_NC, ep, 128), i32),
                  jax.ShapeDtypeStruct((_NC, ep, 128), i32)],
        mesh=mesh,
        scratch_types=[
            pltpu.VMEM((_SB, _CHA), i32),
            pltpu.VMEM((_SB, _CHA), i32),
            nbuf(lambda: pltpu.VMEM((_CHA,), i32)),
            nbuf(lambda: pltpu.VMEM((_CHA,), i32)),
            nbuf(lambda: pltpu.VMEM((_CHA, 128), i32)),
            nbuf(lambda: pltpu.VMEM((_CHA, 128), i32)),
            nbuf(lambda: pltpu.SemaphoreType.DMA),
            nbuf(lambda: pltpu.SemaphoreType.DMA),
            nbuf(lambda: pltpu.SemaphoreType.DMA),
            nbuf(lambda: pltpu.SemaphoreType.DMA),
        ],
        name="gather_sc",
    )(td, ts, dst2, src2)
    return sd, ss


def _scatter_sc_call(m, dst2, n_nodes, ep):
    ep_tile = ep // _NS
    nch = ep_tile // _CHC
    n_pad = -(-n_nodes // (8 * _NS)) * (8 * _NS)
    mesh = plsc.VectorSubcoreMesh(core_axis_name="c", subcore_axis_name="s")
    body = functools.partial(_scatter_sc_body, n_pad=n_pad,
                             ep_tile=ep_tile, nch=nch)

    def two(mk):
        return (mk(), mk())

    agg = pl.kernel(
        body,
        out_type=jax.ShapeDtypeStruct((_NC, n_pad, 128), jnp.float32),
        mesh=mesh,
        scratch_types=[
            pltpu.VMEM((_SB, _CHC), jnp.int32),
            two(lambda: pltpu.VMEM((_CHC,), jnp.int32)),
            two(lambda: pltpu.VMEM((_CHC, 128), jnp.float32)),
            pltpu.VMEM_SHARED((n_pad, 128), jnp.float32),
            two(lambda: pltpu.SemaphoreType.DMA),
        ],
        name="scatter_sc",
    )(m, dst2)
    return agg[:, :n_nodes]


def _perm_w(wf, ws):
    return jnp.concatenate(
        [wf[:, :128], ws[:, :128], wf[:, 128:], ws[:, 128:]], axis=1)


def _perm_b(bf, bs):
    return jnp.concatenate([bf[:128], bs[:128], bf[128:], bs[128:]])[None, :]


def kernel(x, edge_attr, params, edge_index, batch):
    n, dd = x.shape
    e = edge_index.shape[1]
    h_dim = params['Wi'].shape[1]
    out_dim = params['Wo'].shape[1]
    n_layers = len(params['Wf'])
    g = 64

    q = _NS * _SB * max(_CHA * _NBUF, _CHC * 2)
    ep = ((e + q - 1) // q) * q
    dst_p = jnp.concatenate(
        [edge_index[1], jnp.zeros((ep - e,), jnp.int32)]).astype(jnp.int32)
    src_p = jnp.concatenate(
        [edge_index[0], jnp.zeros((ep - e,), jnp.int32)]).astype(jnp.int32)
    dst_a = dst_p.reshape(-1, _CHA)
    src_a = src_p.reshape(-1, _CHA)
    dst_c = dst_p.reshape(-1, _CHC)
    ea_p = jnp.concatenate(
        [edge_attr, jnp.zeros((ep - e, edge_attr.shape[1]), jnp.float32)])

    wd = [_perm_w(params['Wf'][l][:h_dim], params['Ws'][l][:h_dim])
          for l in range(n_layers)]
    wsr = [_perm_w(params['Wf'][l][h_dim:2 * h_dim], params['Ws'][l][h_dim:2 * h_dim])
           for l in range(n_layers)]
    bd = [_perm_b(params['bf'][l], params['bs'][l]) for l in range(n_layers)]
    wa = [_perm_w(params['Wf'][l][2 * h_dim:], params['Ws'][l][2 * h_dim:])
          for l in range(n_layers)]

    nb = 1000
    n_grid = n // nb
    eb = 2048
    e_grid = ep // eb

    f32 = jnp.float32

    h, td, ts = pl.pallas_call(
        _h0_tables_body,
        grid=(n_grid,),
        in_specs=[
            pl.BlockSpec((nb, dd), lambda i: (i, 0)),
            pl.BlockSpec((dd, h_dim), lambda i: (0, 0)),
            pl.BlockSpec((1, h_dim), lambda i: (0, 0)),
            pl.BlockSpec((h_dim, 2 * h_dim), lambda i: (0, 0)),
            pl.BlockSpec((h_dim, 2 * h_dim), lambda i: (0, 0)),
            pl.BlockSpec((1, 2 * h_dim), lambda i: (0, 0)),
        ],
        out_specs=[
            pl.BlockSpec((nb, h_dim), lambda i: (i, 0)),
            pl.BlockSpec((_NC, nb, 128), lambda i: (0, i, 0)),
            pl.BlockSpec((_NC, nb, 128), lambda i: (0, i, 0)),
        ],
        out_shape=[
            jax.ShapeDtypeStruct((n, h_dim), f32),
            jax.ShapeDtypeStruct((_NC, n, 128), jnp.int32),
            jax.ShapeDtypeStruct((_NC, n, 128), jnp.int32),
        ],
    )(x, params['Wi'], params['bi'][None, :], wd[0], wsr[0], bd[0])

    for l in range(n_layers):
        sd, ss = _gather_sc_call(td, ts, dst_a, src_a, ep)

        m = pl.pallas_call(
            functools.partial(_msg_body, blk=eb, e_real=e),
            grid=(e_grid,),
            in_specs=[
                pl.BlockSpec((_NC, eb, 128), lambda i: (0, i, 0)),
                pl.BlockSpec((_NC, eb, 128), lambda i: (0, i, 0)),
                pl.BlockSpec((eb, ea_p.shape[1]), lambda i: (i, 0)),
                pl.BlockSpec((ea_p.shape[1], 2 * h_dim), lambda i: (0, 0)),
            ],
            out_specs=pl.BlockSpec((_NC, eb, 128), lambda i: (0, i, 0)),
            out_shape=jax.ShapeDtypeStruct((_NC, ep, 128), f32),
        )(sd, ss, ea_p, wa[l])

        agg = _scatter_sc_call(m, dst_c, n, ep)

        if l + 1 < n_layers:
            h, td, ts = pl.pallas_call(
                _update_tables_body,
                grid=(n_grid,),
                in_specs=[
                    pl.BlockSpec((_NC, nb, 128), lambda i: (0, i, 0)),
                    pl.BlockSpec((nb, h_dim), lambda i: (i, 0)),
                    pl.BlockSpec((1, h_dim), lambda i: (0, 0)),
                    pl.BlockSpec((1, h_dim), lambda i: (0, 0)),
                    pl.BlockSpec((h_dim, 2 * h_dim), lambda i: (0, 0)),
                    pl.BlockSpec((h_dim, 2 * h_dim), lambda i: (0, 0)),
                    pl.BlockSpec((1, 2 * h_dim), lambda i: (0, 0)),
                ],
                out_specs=[
                    pl.BlockSpec((nb, h_dim), lambda i: (i, 0)),
                    pl.BlockSpec((_NC, nb, 128), lambda i: (0, i, 0)),
                    pl.BlockSpec((_NC, nb, 128), lambda i: (0, i, 0)),
                ],
                out_shape=[
                    jax.ShapeDtypeStruct((n, h_dim), f32),
                    jax.ShapeDtypeStruct((_NC, n, 128), jnp.int32),
                    jax.ShapeDtypeStruct((_NC, n, 128), jnp.int32),
                ],
            )(agg, h, params['ln_g'][l][None, :], params['ln_b'][l][None, :],
              wd[l + 1], wsr[l + 1], bd[l + 1])
        else:
            h = pl.pallas_call(
                _update_only_body,
                grid=(n_grid,),
                in_specs=[
                    pl.BlockSpec((_NC, nb, 128), lambda i: (0, i, 0)),
                    pl.BlockSpec((nb, h_dim), lambda i: (i, 0)),
                    pl.BlockSpec((1, h_dim), lambda i: (0, 0)),
                    pl.BlockSpec((1, h_dim), lambda i: (0, 0)),
                ],
                out_specs=pl.BlockSpec((nb, h_dim), lambda i: (i, 0)),
                out_shape=jax.ShapeDtypeStruct((n, h_dim), f32),
            )(agg, h, params['ln_g'][l][None, :], params['ln_b'][l][None, :])

    out = pl.pallas_call(
        functools.partial(_pool_body, nblk=n_grid, blk=nb, g=g),
        grid=(n_grid,),
        in_specs=[
            pl.BlockSpec((nb, 1), lambda i: (i, 0)),
            pl.BlockSpec((nb, h_dim), lambda i: (i, 0)),
            pl.BlockSpec((h_dim, out_dim), lambda i: (0, 0)),
            pl.BlockSpec((1, out_dim), lambda i: (0, 0)),
        ],
        out_specs=pl.BlockSpec((g, out_dim), lambda i: (0, 0)),
        out_shape=jax.ShapeDtypeStruct((g, out_dim), f32),
        scratch_shapes=[
            pltpu.VMEM((g, h_dim), f32),
            pltpu.VMEM((g, h_dim), f32),
        ],
    )(batch[:, None], h, params['Wo'], params['bo'][None, :])

    return out
